# baseline (speedup 1.0000x reference)
import jax
import jax.numpy as jnp
from jax import lax
from jax.experimental import pallas as pl
from jax.experimental.pallas import tpu as pltpu
from jax.experimental.pallas import tpu_sc as plsc

_NUM_TABLES = 26
_B = 4096
_V = 100000
_D = 64
_NUM_CORES = 2
_NUM_SUBCORES = 16
_NW = _NUM_CORES * _NUM_SUBCORES
_BPW = _B // _NW
_CCH = 16384


def _comp_body(wta_ref, wtb_ref, out_ref):
    out_ref[:, 0:_D] = wta_ref[...].T
    out_ref[:, _D:2 * _D] = wtb_ref[...].T


@jax.jit
def _tc_compact2(wa, wb):
    return pl.pallas_call(
        _comp_body,
        grid=(_V // _CCH + (1 if _V % _CCH else 0),),
        in_specs=[pl.BlockSpec((_D, _CCH), lambda c: (0, c)),
                  pl.BlockSpec((_D, _CCH), lambda c: (0, c))],
        out_specs=pl.BlockSpec((_CCH, 2 * _D), lambda c: (c, 0)),
        out_shape=jax.ShapeDtypeStruct((_V, 2 * _D), jnp.float32),
    )(wa.T, wb.T)


_NBUF = 3


def _sc_body(idx_flat, *rest):
    combs = rest[:_NUM_TABLES // 2]
    ws = [combs[i // 2] for i in range(_NUM_TABLES)]
    outs = rest[_NUM_TABLES // 2:_NUM_TABLES // 2 + _NUM_TABLES]
    scratch = rest[_NUM_TABLES // 2 + _NUM_TABLES:]
    idx_v = scratch[0]
    sem_i = scratch[1]
    bufs = scratch[2:2 + _NBUF]
    gsems = scratch[2 + _NBUF:2 + 2 * _NBUF]
    wsems = scratch[2 + 2 * _NBUF:2 + 3 * _NBUF]

    wid = lax.axis_index("s") * _NUM_CORES + lax.axis_index("c")
    base = wid * _BPW

    stages = [
        pltpu.async_copy(idx_flat.at[pl.ds(i * _B + base, _BPW)],
                         idx_v.at[i], sem_i)
        for i in range(_NUM_TABLES)
    ]
    for st in stages:
        st.wait()

    gathers = [None] * _NBUF
    writes = [None] * _NBUF
    for j in range(_NBUF - 1):
        gathers[j] = pltpu.async_copy(
            ws[j].at[idx_v.at[j]], bufs[j], gsems[j])
    for i in range(_NUM_TABLES):
        b = i % _NBUF
        nxt = i + _NBUF - 1
        if nxt < _NUM_TABLES:
            nb = nxt % _NBUF
            if writes[nb] is not None:
                writes[nb].wait()
                writes[nb] = None
            gathers[nb] = pltpu.async_copy(
                ws[nxt].at[idx_v.at[nxt]], bufs[nb], gsems[nb])
        gathers[b].wait()
        writes[b] = pltpu.async_copy(
            bufs[b], outs[i].at[pl.ds(base, _BPW)], wsems[b])
    for j in range(_NBUF):
        if writes[j] is not None:
            writes[j].wait()


@jax.jit
def _sc_call(idx_flat, *ws):
    mesh = plsc.VectorSubcoreMesh(
        core_axis_name="c", subcore_axis_name="s",
        num_cores=_NUM_CORES, num_subcores=_NUM_SUBCORES)
    return pl.kernel(
        _sc_body,
        out_type=[jax.ShapeDtypeStruct((_B, 2 * _D), jnp.float32)
                  for _ in range(_NUM_TABLES)],
        mesh=mesh,
        scratch_types=(
            [pltpu.VMEM((_NUM_TABLES, _BPW), jnp.int32),
             pltpu.SemaphoreType.DMA]
            + [pltpu.VMEM((_BPW, 2 * _D), jnp.float32)
               for _ in range(_NBUF)]
            + [pltpu.SemaphoreType.DMA for _ in range(2 * _NBUF)]
        ),
    )(idx_flat, *ws)


_TC_ROWS = 512


def _tc_body(dense_ref, *refs):
    pooled = refs[:_NUM_TABLES]
    out_ref = refs[_NUM_TABLES]
    out_ref[:, 0:_D] = dense_ref[...]
    for i in range(_NUM_TABLES):
        h = i % 2
        out_ref[:, (i + 1) * _D:(i + 2) * _D] = pooled[i][:, h * _D:(h + 1) * _D]


@jax.jit
def _tc_finalize(dense, *pooled):
    grid = (_B // _TC_ROWS,)
    return pl.pallas_call(
        _tc_body,
        grid=grid,
        in_specs=(
            [pl.BlockSpec((_TC_ROWS, _D), lambda r: (r, 0))]
            + [pl.BlockSpec((_TC_ROWS, 2 * _D), lambda r: (r, 0))
               for _ in range(_NUM_TABLES)]
        ),
        out_specs=pl.BlockSpec((_TC_ROWS, (_NUM_TABLES + 1) * _D),
                               lambda r: (r, 0)),
        out_shape=jax.ShapeDtypeStruct((_B, (_NUM_TABLES + 1) * _D),
                                       jnp.float32),
    )(dense, *pooled)


def kernel(dense,
           index_0, offset_0, W_0, index_1, offset_1, W_1,
           index_2, offset_2, W_2, index_3, offset_3, W_3,
           index_4, offset_4, W_4, index_5, offset_5, W_5,
           index_6, offset_6, W_6, index_7, offset_7, W_7,
           index_8, offset_8, W_8, index_9, offset_9, W_9,
           index_10, offset_10, W_10, index_11, offset_11, W_11,
           index_12, offset_12, W_12, index_13, offset_13, W_13,
           index_14, offset_14, W_14, index_15, offset_15, W_15,
           index_16, offset_16, W_16, index_17, offset_17, W_17,
           index_18, offset_18, W_18, index_19, offset_19, W_19,
           index_20, offset_20, W_20, index_21, offset_21, W_21,
           index_22, offset_22, W_22, index_23, offset_23, W_23,
           index_24, offset_24, W_24, index_25, offset_25, W_25):
    del offset_0, offset_1, offset_2, offset_3, offset_4, offset_5
    del offset_6, offset_7, offset_8, offset_9, offset_10, offset_11
    del offset_12, offset_13, offset_14, offset_15, offset_16, offset_17
    del offset_18, offset_19, offset_20, offset_21, offset_22, offset_23
    del offset_24, offset_25
    idxs = [index_0, index_1, index_2, index_3, index_4, index_5, index_6,
            index_7, index_8, index_9, index_10, index_11, index_12,
            index_13, index_14, index_15, index_16, index_17, index_18,
            index_19, index_20, index_21, index_22, index_23, index_24,
            index_25]
    ws = [W_0, W_1, W_2, W_3, W_4, W_5, W_6, W_7, W_8, W_9, W_10, W_11,
          W_12, W_13, W_14, W_15, W_16, W_17, W_18, W_19, W_20, W_21,
          W_22, W_23, W_24, W_25]
    idx_flat = jnp.concatenate(idxs, axis=0)
    wp = [_tc_compact2(ws[2 * k], ws[2 * k + 1])
          for k in range(_NUM_TABLES // 2)]
    pooled = _sc_call(idx_flat, *wp)
    return _tc_finalize(dense, *pooled)


# speedup vs baseline: 1.0651x; 1.0651x over previous
import jax
import jax.numpy as jnp
from jax import lax
from jax.experimental import pallas as pl
from jax.experimental.pallas import tpu as pltpu
from jax.experimental.pallas import tpu_sc as plsc

_NUM_TABLES = 26
_B = 4096
_V = 100000
_D = 64
_NUM_CORES = 2
_NUM_SUBCORES = 16
_NW = _NUM_CORES * _NUM_SUBCORES
_BPW = _B // _NW
_CCH = 8192


def _comp_body(wta_ref, wtb_ref, out_ref):
    out_ref[:, 0:_D] = wta_ref[...].T
    out_ref[:, _D:2 * _D] = wtb_ref[...].T


@jax.jit
def _tc_compact2(wa, wb):
    return pl.pallas_call(
        _comp_body,
        grid=(_V // _CCH + (1 if _V % _CCH else 0),),
        in_specs=[pl.BlockSpec((_D, _CCH), lambda c: (0, c)),
                  pl.BlockSpec((_D, _CCH), lambda c: (0, c))],
        out_specs=pl.BlockSpec((_CCH, 2 * _D), lambda c: (c, 0)),
        out_shape=jax.ShapeDtypeStruct((_V, 2 * _D), jnp.float32),
    )(wa.T, wb.T)


_NBUF = 3


def _sc_body(idx_flat, *rest):
    combs = rest[:_NUM_TABLES // 2]
    ws = [combs[i // 2] for i in range(_NUM_TABLES)]
    outs = rest[_NUM_TABLES // 2:_NUM_TABLES // 2 + _NUM_TABLES]
    scratch = rest[_NUM_TABLES // 2 + _NUM_TABLES:]
    idx_v = scratch[0]
    sem_i = scratch[1]
    bufs = scratch[2:2 + _NBUF]
    gsems = scratch[2 + _NBUF:2 + 2 * _NBUF]
    wsems = scratch[2 + 2 * _NBUF:2 + 3 * _NBUF]

    wid = lax.axis_index("s") * _NUM_CORES + lax.axis_index("c")
    base = wid * _BPW

    stages = [
        pltpu.async_copy(idx_flat.at[pl.ds(i * _B + base, _BPW)],
                         idx_v.at[i], sem_i)
        for i in range(_NUM_TABLES)
    ]
    for st in stages:
        st.wait()

    gathers = [None] * _NBUF
    writes = [None] * _NBUF
    for j in range(_NBUF - 1):
        gathers[j] = pltpu.async_copy(
            ws[j].at[idx_v.at[j]], bufs[j], gsems[j])
    for i in range(_NUM_TABLES):
        b = i % _NBUF
        nxt = i + _NBUF - 1
        if nxt < _NUM_TABLES:
            nb = nxt % _NBUF
            if writes[nb] is not None:
                writes[nb].wait()
                writes[nb] = None
            gathers[nb] = pltpu.async_copy(
                ws[nxt].at[idx_v.at[nxt]], bufs[nb], gsems[nb])
        gathers[b].wait()
        writes[b] = pltpu.async_copy(
            bufs[b], outs[i].at[pl.ds(base, _BPW)], wsems[b])
    for j in range(_NBUF):
        if writes[j] is not None:
            writes[j].wait()


@jax.jit
def _sc_call(idx_flat, *ws):
    mesh = plsc.VectorSubcoreMesh(
        core_axis_name="c", subcore_axis_name="s",
        num_cores=_NUM_CORES, num_subcores=_NUM_SUBCORES)
    return pl.kernel(
        _sc_body,
        out_type=[jax.ShapeDtypeStruct((_B, 2 * _D), jnp.float32)
                  for _ in range(_NUM_TABLES)],
        mesh=mesh,
        scratch_types=(
            [pltpu.VMEM((_NUM_TABLES, _BPW), jnp.int32),
             pltpu.SemaphoreType.DMA]
            + [pltpu.VMEM((_BPW, 2 * _D), jnp.float32)
               for _ in range(_NBUF)]
            + [pltpu.SemaphoreType.DMA for _ in range(2 * _NBUF)]
        ),
    )(idx_flat, *ws)


_TC_ROWS = 512


def _tc_body(dense_ref, *refs):
    pooled = refs[:_NUM_TABLES]
    out_ref = refs[_NUM_TABLES]
    out_ref[0:_D, :] = dense_ref[...].T
    for i in range(_NUM_TABLES):
        h = i % 2
        out_ref[(i + 1) * _D:(i + 2) * _D, :] = (
            pooled[i][:, h * _D:(h + 1) * _D].T)


@jax.jit
def _tc_finalize(dense, *pooled):
    grid = (_B // _TC_ROWS,)
    out_t = pl.pallas_call(
        _tc_body,
        grid=grid,
        in_specs=(
            [pl.BlockSpec((_TC_ROWS, _D), lambda r: (r, 0))]
            + [pl.BlockSpec((_TC_ROWS, 2 * _D), lambda r: (r, 0))
               for _ in range(_NUM_TABLES)]
        ),
        out_specs=pl.BlockSpec(((_NUM_TABLES + 1) * _D, _TC_ROWS),
                               lambda r: (0, r)),
        out_shape=jax.ShapeDtypeStruct(((_NUM_TABLES + 1) * _D, _B),
                                       jnp.float32),
    )(dense, *pooled)
    return out_t.T


def kernel(dense,
           index_0, offset_0, W_0, index_1, offset_1, W_1,
           index_2, offset_2, W_2, index_3, offset_3, W_3,
           index_4, offset_4, W_4, index_5, offset_5, W_5,
           index_6, offset_6, W_6, index_7, offset_7, W_7,
           index_8, offset_8, W_8, index_9, offset_9, W_9,
           index_10, offset_10, W_10, index_11, offset_11, W_11,
           index_12, offset_12, W_12, index_13, offset_13, W_13,
           index_14, offset_14, W_14, index_15, offset_15, W_15,
           index_16, offset_16, W_16, index_17, offset_17, W_17,
           index_18, offset_18, W_18, index_19, offset_19, W_19,
           index_20, offset_20, W_20, index_21, offset_21, W_21,
           index_22, offset_22, W_22, index_23, offset_23, W_23,
           index_24, offset_24, W_24, index_25, offset_25, W_25):
    del offset_0, offset_1, offset_2, offset_3, offset_4, offset_5
    del offset_6, offset_7, offset_8, offset_9, offset_10, offset_11
    del offset_12, offset_13, offset_14, offset_15, offset_16, offset_17
    del offset_18, offset_19, offset_20, offset_21, offset_22, offset_23
    del offset_24, offset_25
    idxs = [index_0, index_1, index_2, index_3, index_4, index_5, index_6,
            index_7, index_8, index_9, index_10, index_11, index_12,
            index_13, index_14, index_15, index_16, index_17, index_18,
            index_19, index_20, index_21, index_22, index_23, index_24,
            index_25]
    ws = [W_0, W_1, W_2, W_3, W_4, W_5, W_6, W_7, W_8, W_9, W_10, W_11,
          W_12, W_13, W_14, W_15, W_16, W_17, W_18, W_19, W_20, W_21,
          W_22, W_23, W_24, W_25]
    idx_flat = jnp.concatenate(idxs, axis=0)
    wp = [_tc_compact2(ws[2 * k], ws[2 * k + 1])
          for k in range(_NUM_TABLES // 2)]
    pooled = _sc_call(idx_flat, *wp)
    return _tc_finalize(dense, *pooled)


# speedup vs baseline: 1.0740x; 1.0083x over previous
import jax
import jax.numpy as jnp
from jax import lax
from jax.experimental import pallas as pl
from jax.experimental.pallas import tpu as pltpu
from jax.experimental.pallas import tpu_sc as plsc

_NUM_TABLES = 26
_B = 4096
_V = 100000
_D = 64
_NUM_CORES = 2
_NUM_SUBCORES = 16
_NW = _NUM_CORES * _NUM_SUBCORES
_BPW = _B // _NW
_CCH = 8192


def _comp_body(wta_ref, wtb_ref, out_ref):
    out_ref[:, 0:_D] = wta_ref[...].T
    out_ref[:, _D:2 * _D] = wtb_ref[...].T


@jax.jit
def _tc_compact2(wa, wb):
    return pl.pallas_call(
        _comp_body,
        grid=(_V // _CCH + (1 if _V % _CCH else 0),),
        in_specs=[pl.BlockSpec((_D, _CCH), lambda c: (0, c)),
                  pl.BlockSpec((_D, _CCH), lambda c: (0, c))],
        out_specs=pl.BlockSpec((_CCH, 2 * _D), lambda c: (c, 0)),
        out_shape=jax.ShapeDtypeStruct((_V, 2 * _D), jnp.float32),
    )(wa.T, wb.T)


_NBUF = 3


def _sc_body(n_tables, idx_flat, *rest):
    combs = rest[:n_tables // 2]
    ws = [combs[i // 2] for i in range(n_tables)]
    outs = rest[n_tables // 2:n_tables // 2 + n_tables]
    scratch = rest[n_tables // 2 + n_tables:]
    idx_v = scratch[0]
    sem_i = scratch[1]
    bufs = scratch[2:2 + _NBUF]
    gsems = scratch[2 + _NBUF:2 + 2 * _NBUF]
    wsems = scratch[2 + 2 * _NBUF:2 + 3 * _NBUF]

    wid = lax.axis_index("s") * _NUM_CORES + lax.axis_index("c")
    base = wid * _BPW

    stages = [
        pltpu.async_copy(idx_flat.at[pl.ds(i * _B + base, _BPW)],
                         idx_v.at[i], sem_i)
        for i in range(n_tables)
    ]
    for st in stages:
        st.wait()

    gathers = [None] * _NBUF
    writes = [None] * _NBUF
    for j in range(_NBUF - 1):
        gathers[j] = pltpu.async_copy(
            ws[j].at[idx_v.at[j]], bufs[j], gsems[j])
    for i in range(n_tables):
        b = i % _NBUF
        nxt = i + _NBUF - 1
        if nxt < n_tables:
            nb = nxt % _NBUF
            if writes[nb] is not None:
                writes[nb].wait()
                writes[nb] = None
            gathers[nb] = pltpu.async_copy(
                ws[nxt].at[idx_v.at[nxt]], bufs[nb], gsems[nb])
        gathers[b].wait()
        writes[b] = pltpu.async_copy(
            bufs[b], outs[i].at[pl.ds(base, _BPW)], wsems[b])
    for j in range(_NBUF):
        if writes[j] is not None:
            writes[j].wait()


def _sc_call(idx_flat, *ws):
    import functools
    n_tables = 2 * len(ws)
    mesh = plsc.VectorSubcoreMesh(
        core_axis_name="c", subcore_axis_name="s",
        num_cores=_NUM_CORES, num_subcores=_NUM_SUBCORES)
    return pl.kernel(
        functools.partial(_sc_body, n_tables),
        out_type=[jax.ShapeDtypeStruct((_B, 2 * _D), jnp.float32)
                  for _ in range(n_tables)],
        mesh=mesh,
        scratch_types=(
            [pltpu.VMEM((n_tables, _BPW), jnp.int32),
             pltpu.SemaphoreType.DMA]
            + [pltpu.VMEM((_BPW, 2 * _D), jnp.float32)
               for _ in range(_NBUF)]
            + [pltpu.SemaphoreType.DMA for _ in range(2 * _NBUF)]
        ),
    )(idx_flat, *ws)


_TC_ROWS = 512


def _tc_body(dense_ref, *refs):
    pooled = refs[:_NUM_TABLES]
    out_ref = refs[_NUM_TABLES]
    out_ref[0:_D, :] = dense_ref[...].T
    for i in range(_NUM_TABLES):
        h = i % 2
        out_ref[(i + 1) * _D:(i + 2) * _D, :] = (
            pooled[i][:, h * _D:(h + 1) * _D].T)


@jax.jit
def _tc_finalize(dense, *pooled):
    grid = (_B // _TC_ROWS,)
    out_t = pl.pallas_call(
        _tc_body,
        grid=grid,
        in_specs=(
            [pl.BlockSpec((_TC_ROWS, _D), lambda r: (r, 0))]
            + [pl.BlockSpec((_TC_ROWS, 2 * _D), lambda r: (r, 0))
               for _ in range(_NUM_TABLES)]
        ),
        out_specs=pl.BlockSpec(((_NUM_TABLES + 1) * _D, _TC_ROWS),
                               lambda r: (0, r)),
        out_shape=jax.ShapeDtypeStruct(((_NUM_TABLES + 1) * _D, _B),
                                       jnp.float32),
    )(dense, *pooled)
    return out_t.T


def kernel(dense,
           index_0, offset_0, W_0, index_1, offset_1, W_1,
           index_2, offset_2, W_2, index_3, offset_3, W_3,
           index_4, offset_4, W_4, index_5, offset_5, W_5,
           index_6, offset_6, W_6, index_7, offset_7, W_7,
           index_8, offset_8, W_8, index_9, offset_9, W_9,
           index_10, offset_10, W_10, index_11, offset_11, W_11,
           index_12, offset_12, W_12, index_13, offset_13, W_13,
           index_14, offset_14, W_14, index_15, offset_15, W_15,
           index_16, offset_16, W_16, index_17, offset_17, W_17,
           index_18, offset_18, W_18, index_19, offset_19, W_19,
           index_20, offset_20, W_20, index_21, offset_21, W_21,
           index_22, offset_22, W_22, index_23, offset_23, W_23,
           index_24, offset_24, W_24, index_25, offset_25, W_25):
    del offset_0, offset_1, offset_2, offset_3, offset_4, offset_5
    del offset_6, offset_7, offset_8, offset_9, offset_10, offset_11
    del offset_12, offset_13, offset_14, offset_15, offset_16, offset_17
    del offset_18, offset_19, offset_20, offset_21, offset_22, offset_23
    del offset_24, offset_25
    idxs = [index_0, index_1, index_2, index_3, index_4, index_5, index_6,
            index_7, index_8, index_9, index_10, index_11, index_12,
            index_13, index_14, index_15, index_16, index_17, index_18,
            index_19, index_20, index_21, index_22, index_23, index_24,
            index_25]
    ws = [W_0, W_1, W_2, W_3, W_4, W_5, W_6, W_7, W_8, W_9, W_10, W_11,
          W_12, W_13, W_14, W_15, W_16, W_17, W_18, W_19, W_20, W_21,
          W_22, W_23, W_24, W_25]
    split_pairs = 7
    split_tables = 2 * split_pairs
    idx_a = jnp.concatenate(idxs[:split_tables], axis=0)
    idx_b = jnp.concatenate(idxs[split_tables:], axis=0)
    wp_a = [_tc_compact2(ws[2 * k], ws[2 * k + 1])
            for k in range(split_pairs)]
    wp_b = [_tc_compact2(ws[2 * k], ws[2 * k + 1])
            for k in range(split_pairs, _NUM_TABLES // 2)]
    pooled_a = _sc_call(idx_a, *wp_a)
    pooled_b = _sc_call(idx_b, *wp_b)
    return _tc_finalize(dense, *(list(pooled_a) + list(pooled_b)))
